# baseline (speedup 1.0000x reference)
import functools

import jax
import jax.numpy as jnp
from jax.experimental import pallas as pl
from jax.experimental.pallas import tpu as pltpu
from jax.experimental.pallas import tpu_sc as plsc

_CHUNK = 8192


def _dot(a, b, dims):
    return jax.lax.dot_general(
        a, b, (dims, ((), ())),
        precision=jax.lax.Precision.HIGHEST,
        preferred_element_type=jnp.float32,
    )


def _rank_body(x_ref, w_ref, o1_ref, o2_ref, bv1_ref, bi1_ref, bv2_ref, bi2_ref):
    i = pl.program_id(0)
    c = w_ref.shape[1]

    @pl.when(i == 0)
    def _init():
        bv1_ref[...] = jnp.full(bv1_ref.shape, jnp.inf, jnp.float32)
        bi1_ref[...] = jnp.zeros(bi1_ref.shape, jnp.int32)
        bv2_ref[...] = jnp.full(bv2_ref.shape, jnp.inf, jnp.float32)
        bi2_ref[...] = jnp.zeros(bi2_ref.shape, jnp.int32)

    x = x_ref[...]
    wb = w_ref[0]

    wsqc = jnp.sum(wb * wb, axis=1, keepdims=True)
    ones = jnp.ones((x.shape[0], 1), jnp.float32)
    wsqb = _dot(ones, wsqc, ((1,), (1,)))
    s2 = _dot(-2.0 * x, wb, ((1,), (1,)))
    dist = wsqb + s2

    iota = jax.lax.broadcasted_iota(jnp.int32, dist.shape, 1)
    big = jnp.int32(c)
    m1 = jnp.min(dist, axis=1, keepdims=True)
    idx1 = jnp.min(jnp.where(dist == m1, iota, big), axis=1, keepdims=True)
    distm = jnp.where(iota == idx1, jnp.inf, dist)
    m2 = jnp.min(distm, axis=1, keepdims=True)
    idx2 = jnp.min(jnp.where(distm == m2, iota, big), axis=1, keepdims=True)
    g1 = idx1 + i * c
    g2 = idx2 + i * c

    bv1 = bv1_ref[...]
    bi1 = bi1_ref[...]
    bv2 = bv2_ref[...]
    bi2 = bi2_ref[...]
    t = m1 < bv1
    nv1 = jnp.where(t, m1, bv1)
    ni1 = jnp.where(t, g1, bi1)
    lv = jnp.where(t, bv1, m1)
    li = jnp.where(t, bi1, g1)
    u = bv2 < m2
    cv = jnp.where(u, bv2, m2)
    ci = jnp.where(u, bi2, g2)
    s = lv < cv
    nv2 = jnp.where(s, lv, cv)
    ni2 = jnp.where(s, li, ci)
    bv1_ref[...] = nv1
    bi1_ref[...] = ni1
    bv2_ref[...] = nv2
    bi2_ref[...] = ni2

    @pl.when(i == pl.num_programs(0) - 1)
    def _finish():
        o1_ref[...] = ni1
        o2_ref[...] = ni2


def _tc_rank(x, w3d):
    _, p, d = w3d.shape
    b = x.shape[0]
    n_chunks = p // _CHUNK
    return pl.pallas_call(
        _rank_body,
        grid=(n_chunks,),
        in_specs=[
            pl.BlockSpec((b, d), lambda i: (0, 0)),
            pl.BlockSpec((1, _CHUNK, d), lambda i: (0, i, 0)),
        ],
        out_specs=[
            pl.BlockSpec((b, 1), lambda i: (0, 0)),
            pl.BlockSpec((b, 1), lambda i: (0, 0)),
        ],
        out_shape=[
            jax.ShapeDtypeStruct((b, 1), jnp.int32),
            jax.ShapeDtypeStruct((b, 1), jnp.int32),
        ],
        scratch_shapes=[
            pltpu.VMEM((b, 1), jnp.float32),
            pltpu.VMEM((b, 1), jnp.int32),
            pltpu.VMEM((b, 1), jnp.float32),
            pltpu.VMEM((b, 1), jnp.int32),
        ],
    )(x, w3d)


def _sc_finish(xt, w1, g1, cand64, b, d):
    n = cand64.shape[0]
    nrow = (n * d) // 128
    mesh = plsc.VectorSubcoreMesh(core_axis_name="c", subcore_axis_name="s")

    @functools.partial(
        pl.kernel,
        mesh=mesh,
        out_type=jax.ShapeDtypeStruct((n,), jnp.float32),
        scratch_types=[
            pltpu.VMEM((n,), jnp.int32),
            pltpu.VMEM((b * d,), jnp.float32),
            pltpu.VMEM((nrow, 128), jnp.int32),
            pltpu.VMEM((nrow, 128), jnp.float32),
            pltpu.VMEM((n,), jnp.float32),
            pltpu.VMEM((n,), jnp.int32),
            pltpu.VMEM((n,), jnp.float32),
            pltpu.SemaphoreType.DMA,
        ],
    )
    def finish_k(xt_hbm, w_hbm, g_hbm, cand_hbm, out_hbm,
                 idx_v, x_v, wi_v, wv_v, dd_v, gi_v, gc_v, sem):
        wid = jax.lax.axis_index("s") * 2 + jax.lax.axis_index("c")

        @pl.when(wid == 0)
        def _():
            pltpu.sync_copy(cand_hbm, idx_v)
            pltpu.sync_copy(xt_hbm, x_v)
            for dd in range(d):
                for g in range(n // 16):
                    p = dd * n + g * 16
                    cv = idx_v[pl.ds(g * 16, 16)]
                    wi_v[p // 128, pl.ds(p % 128, 16)] = cv * d + dd
            cps = [pltpu.async_copy(w_hbm.at[wi_v.at[r]], wv_v.at[r], sem)
                   for r in range(nrow)]
            for cp in cps:
                cp.wait()
            for g in range(n // 16):
                acc = jnp.zeros((16,), jnp.float32)
                xoff = (g % (b // 16)) * 16
                for dd in range(d):
                    p = dd * n + g * 16
                    xg = x_v[pl.ds(dd * b + xoff, 16)]
                    wg = wv_v[p // 128, pl.ds(p % 128, 16)]
                    t = xg - wg
                    acc = acc + t * t
                dd_v[pl.ds(g * 16, 16)] = acc
            for g in range(b // 16):
                d1 = dd_v[pl.ds(g * 16, 16)]
                d2 = dd_v[pl.ds(b + g * 16, 16)]
                i1 = idx_v[pl.ds(g * 16, 16)]
                i2 = idx_v[pl.ds(b + g * 16, 16)]
                pick2 = (d2 < d1) | ((d2 == d1) & (i2 < i1))
                win = jnp.where(pick2, i2, i1)
                gi_v[pl.ds(g * 16, 16)] = win * 2
                gi_v[pl.ds(b + g * 16, 16)] = win * 2 + 1
            pltpu.async_copy(g_hbm.at[gi_v], gc_v, sem).wait()
            pltpu.sync_copy(gc_v, out_hbm)

    return finish_k(xt, w1, g1, cand64)


def kernel(x, grid_flattened, w):
    b, d = x.shape
    c1, c2 = _tc_rank(x, w)
    cand64 = jnp.concatenate([c1.reshape(-1), c2.reshape(-1)])
    xt = x.T.reshape(-1)
    w1 = w.reshape(-1)
    g1 = grid_flattened.reshape(-1)
    vals = _sc_finish(xt, w1, g1, cand64, b, d)
    return vals.reshape(2, b).T
